# baseline (speedup 1.0000x reference)
import functools

import jax
import jax.numpy as jnp
from jax import lax
from jax.experimental import pallas as pl
from jax.experimental.pallas import tpu as pltpu
from jax.experimental.pallas import tpu_sc as plsc

N = 10000
E = 320000
D = 128
H = 128
C = 3

NC = 2
NS = 16
NW = NC * NS
NPAD = 10240
RPT = NPAD // NS
EPT = E // NW
K = 80

BLK = 1000

@functools.lru_cache(maxsize=None)
def _mesh():
    return plsc.VectorSubcoreMesh(core_axis_name="c", subcore_axis_name="s",
                                  num_cores=NC, num_subcores=NS)


def _sc_degree_body(dst_hbm, deg_hbm, idx_v, ones_v, buf_v,
                    deg_sh, isem, ssem0, ssem1):
    c = lax.axis_index("c")
    s = lax.axis_index("s")
    wid = s * NC + c
    pltpu.async_copy(dst_hbm.at[wid], idx_v, isem)
    zero16 = jnp.zeros((16,), jnp.float32)
    one16 = jnp.ones((16,), jnp.float32)
    for i in range(K // 16):
        ones_v[pl.ds(i * 16, 16)] = one16
    for i in range(RPT // 16):
        buf_v[pl.ds(i * 16, 16)] = zero16
    pltpu.sync_copy(buf_v, deg_sh.at[pl.ds(s * RPT, RPT)])
    pltpu.make_async_copy(dst_hbm.at[0], idx_v, isem).wait()
    plsc.subcore_barrier()

    ssem = (ssem0, ssem1)
    NCH = EPT // K

    def sstart(j, p):
        pltpu.async_copy(ones_v, deg_sh.at[idx_v.at[j]], ssem[p], add=True)

    def swait(p):
        pltpu.make_async_copy(ones_v, deg_sh.at[idx_v.at[0]], ssem[p]).wait()

    sstart(0, 0)
    sstart(1, 1)

    def pair(g, carry):
        j0 = 2 * g + 2
        swait(0)
        sstart(j0, 0)
        swait(1)
        sstart(j0 + 1, 1)
        return carry

    lax.fori_loop(0, (NCH - 3) // 2, pair, 0)
    swait(0)
    sstart(124, 0)
    swait(1)
    swait(0)
    plsc.subcore_barrier()
    pltpu.sync_copy(deg_sh.at[pl.ds(s * RPT, RPT)],
                    deg_hbm.at[c, pl.ds(s * RPT, RPT)])


@functools.lru_cache(maxsize=None)
def _sc_degree_kernel():
    return pl.kernel(
        _sc_degree_body,
        out_type=jax.ShapeDtypeStruct((NC, NPAD), jnp.float32),
        mesh=_mesh(),
        scratch_types=[
            pltpu.VMEM((EPT // K, K), jnp.int32),
            pltpu.VMEM((K,), jnp.float32),
            pltpu.VMEM((RPT,), jnp.float32),
            pltpu.VMEM_SHARED((NPAD,), jnp.float32),
            pltpu.SemaphoreType.DMA,
            pltpu.SemaphoreType.DMA,
            pltpu.SemaphoreType.DMA,
        ],
    )


KS = 80
NCHUNK = EPT // KS


def _sc_scatter_body(src_hbm, dst_hbm, hs_hbm, acc_hbm,
                     sidx_v, didx_v, rows0, rows1, acc_sh,
                     isem, gsem0, gsem1, ssem0, ssem1):
    c = lax.axis_index("c")
    s = lax.axis_index("s")
    wid = s * NC + c
    pltpu.async_copy(src_hbm.at[wid], sidx_v, isem)
    pltpu.async_copy(dst_hbm.at[wid], didx_v, isem)
    zero16 = jnp.zeros((16,), jnp.float32)

    def zrow(i, carry):
        for j in range(D // 16):
            rows0[i, pl.ds(j * 16, 16)] = zero16
        return carry

    lax.fori_loop(0, KS, zrow, 0)

    rows = (rows0, rows1)
    gsem = (gsem0, gsem1)
    ssem = (ssem0, ssem1)

    def gather(j, b):
        pltpu.async_copy(hs_hbm.at[sidx_v.at[pl.ds(j * KS, KS)]], rows[b],
                         gsem[b])

    def gwait(b):
        pltpu.make_async_copy(hs_hbm.at[sidx_v.at[pl.ds(0, KS)]], rows[b],
                              gsem[b]).wait()

    def sstart(j, b):
        pltpu.async_copy(rows[b], acc_sh.at[didx_v.at[j]], ssem[b], add=True)

    def swait(b):
        pltpu.make_async_copy(rows[0], acc_sh.at[didx_v.at[0]],
                              ssem[b]).wait()

    pltpu.make_async_copy(src_hbm.at[0], sidx_v, isem).wait()
    pltpu.make_async_copy(dst_hbm.at[0], didx_v, isem).wait()
    gather(0, 1)
    for t in range(RPT // KS):
        pltpu.async_copy(rows0, acc_sh.at[pl.ds(s * RPT + t * KS, KS)],
                         isem)
    for t in range(RPT // KS):
        pltpu.make_async_copy(rows0, acc_sh.at[pl.ds(0, KS)], isem).wait()
    plsc.subcore_barrier()

    def body(j, ph, do_swait=True, do_gather=True):
        if do_swait:
            swait(1 - ph)
        if do_gather:
            gather(j + 1, 1 - ph)
        gwait(ph)
        sstart(j, ph)

    body(0, 1, do_swait=False)

    def pair(g, carry):
        j0 = 2 * g + 1
        body(j0, 0)
        body(j0 + 1, 1)
        return carry

    lax.fori_loop(0, (NCHUNK - 3) // 2, pair, 0)
    body(123, 0)
    body(124, 1, do_gather=False)
    swait(1)
    plsc.subcore_barrier()
    pltpu.sync_copy(acc_sh.at[pl.ds(s * RPT, RPT)],
                    acc_hbm.at[c, pl.ds(s * RPT, RPT)])


@functools.lru_cache(maxsize=None)
def _sc_scatter_kernel():
    return pl.kernel(
        _sc_scatter_body,
        out_type=jax.ShapeDtypeStruct((NC, NPAD, D), jnp.float32),
        mesh=_mesh(),
        scratch_types=[
            pltpu.VMEM((EPT,), jnp.int32),
            pltpu.VMEM((NCHUNK, KS), jnp.int32),
            pltpu.VMEM((KS, D), jnp.float32),
            pltpu.VMEM((KS, D), jnp.float32),
            pltpu.VMEM_SHARED((NPAD, D), jnp.float32),
            pltpu.SemaphoreType.DMA,
            pltpu.SemaphoreType.DMA,
            pltpu.SemaphoreType.DMA,
            pltpu.SemaphoreType.DMA,
            pltpu.SemaphoreType.DMA,
        ],
    )


def _dinv_of(degb):
    return lax.rsqrt(degb[:, 0] + degb[:, 1] + 1.0)


def _tc1_body(x_ref, w_ref, out_ref):
    out_ref[...] = jnp.dot(x_ref[...], w_ref[...],
                           preferred_element_type=jnp.float32)


def _tc1(x, W1):
    return pl.pallas_call(
        _tc1_body,
        grid=(N // BLK,),
        in_specs=[
            pl.BlockSpec((BLK, D), lambda i: (i, 0)),
            pl.BlockSpec((D, H), lambda i: (0, 0)),
        ],
        out_specs=pl.BlockSpec((BLK, H), lambda i: (i, 0)),
        out_shape=jax.ShapeDtypeStruct((N, H), jnp.float32),
    )(x, W1)


def _tc1s_body(xw_ref, deg_ref, out_ref):
    dinv = _dinv_of(deg_ref[...])
    out_ref[...] = xw_ref[...] * dinv[:, None]


def _tc1s(xw, deg2):
    return pl.pallas_call(
        _tc1s_body,
        grid=(N // BLK,),
        in_specs=[
            pl.BlockSpec((BLK, H), lambda i: (i, 0)),
            pl.BlockSpec((BLK, NC), lambda i: (i, 0)),
        ],
        out_specs=pl.BlockSpec((BLK, H), lambda i: (i, 0)),
        out_shape=jax.ShapeDtypeStruct((N, H), jnp.float32),
    )(xw, deg2)


def _ln_relu(t, w, b):
    mu = jnp.mean(t, axis=-1, keepdims=True)
    var = jnp.mean((t - mu) ** 2, axis=-1, keepdims=True)
    t = (t - mu) * lax.rsqrt(var + 1e-5) * w + b
    return jnp.maximum(t, 0.0)


def _tc2_body(hs_ref, acc_ref, deg_ref, b_ref, lnw_ref, lnb_ref, w2_ref,
              out_ref):
    dinv = _dinv_of(deg_ref[...])
    a = acc_ref[0] + acc_ref[1]
    t = (hs_ref[...] + a) * dinv[:, None] + b_ref[...]
    t = _ln_relu(t, lnw_ref[...], lnb_ref[...])
    h2 = jnp.dot(t, w2_ref[...], preferred_element_type=jnp.float32)
    out_ref[...] = h2 * dinv[:, None]


def _tc2(hs1, acc1, deg2, b1, lnw, lnb, W2):
    return pl.pallas_call(
        _tc2_body,
        grid=(N // BLK,),
        in_specs=[
            pl.BlockSpec((BLK, H), lambda i: (i, 0)),
            pl.BlockSpec((NC, BLK, H), lambda i: (0, i, 0)),
            pl.BlockSpec((BLK, NC), lambda i: (i, 0)),
            pl.BlockSpec((1, H), lambda i: (0, 0)),
            pl.BlockSpec((1, H), lambda i: (0, 0)),
            pl.BlockSpec((1, H), lambda i: (0, 0)),
            pl.BlockSpec((H, H), lambda i: (0, 0)),
        ],
        out_specs=pl.BlockSpec((BLK, H), lambda i: (i, 0)),
        out_shape=jax.ShapeDtypeStruct((N, H), jnp.float32),
    )(hs1, acc1, deg2, b1.reshape(1, H), lnw.reshape(1, H),
      lnb.reshape(1, H), W2)


NB = N // BLK


def _tc34_body(hs_ref, acc_ref, deg_ref, b_ref, lnw_ref, lnb_ref,
               wg_ref, bg_ref, wf1_ref, bf1_ref, wf2_ref, bf2_ref,
               out_ref, h_scr, gsum_scr):
    i = pl.program_id(0)

    @pl.when(i < NB)
    def _():
        dinv = _dinv_of(deg_ref[...])
        a = acc_ref[0] + acc_ref[1]
        t = (hs_ref[...] + a) * dinv[:, None] + b_ref[...]
        t = _ln_relu(t, lnw_ref[...], lnb_ref[...])
        h_scr[pl.ds(i * BLK, BLK)] = t

        @pl.when(i == 0)
        def _():
            gsum_scr[...] = jnp.zeros_like(gsum_scr)

        gsum_scr[...] += jnp.sum(t, axis=0, keepdims=True)

    @pl.when(i >= NB)
    def _():
        g = gsum_scr[...] * (1.0 / N)
        tg = jnp.dot(g, wg_ref[...], preferred_element_type=jnp.float32) \
            + bg_ref[...]
        r = h_scr[pl.ds((i - NB) * BLK, BLK)] + tg
        r = jnp.dot(r, wf1_ref[...], preferred_element_type=jnp.float32) \
            + bf1_ref[...]
        r = jnp.maximum(r, 0.0)
        o = jnp.dot(r, wf2_ref[...], preferred_element_type=jnp.float32) \
            + bf2_ref[...]
        out_ref[...] = jnp.tanh(o)


def _tc34(hs2, acc2, deg2, b2, lnw, lnb, Wg, bg, Wf1, bf1, Wf2, bf2):
    lo = lambda i: (jnp.minimum(i, NB - 1), 0)
    lo3 = lambda i: (0, jnp.minimum(i, NB - 1), 0)
    z = lambda i: (0, 0)
    return pl.pallas_call(
        _tc34_body,
        grid=(2 * NB,),
        in_specs=[
            pl.BlockSpec((BLK, H), lo),
            pl.BlockSpec((NC, BLK, H), lo3),
            pl.BlockSpec((BLK, NC), lo),
            pl.BlockSpec((1, H), z),
            pl.BlockSpec((1, H), z),
            pl.BlockSpec((1, H), z),
            pl.BlockSpec((H, H), z),
            pl.BlockSpec((1, H), z),
            pl.BlockSpec((H, H // 2), z),
            pl.BlockSpec((1, H // 2), z),
            pl.BlockSpec((H // 2, C), z),
            pl.BlockSpec((1, C), z),
        ],
        out_specs=pl.BlockSpec((BLK, C), lambda i: (jnp.maximum(i - NB, 0),
                                                    0)),
        out_shape=jax.ShapeDtypeStruct((N, C), jnp.float32),
        scratch_shapes=[
            pltpu.VMEM((N, H), jnp.float32),
            pltpu.VMEM((1, H), jnp.float32),
        ],
    )(hs2, acc2, deg2, b2.reshape(1, H), lnw.reshape(1, H),
      lnb.reshape(1, H), Wg, bg.reshape(1, H), Wf1, bf1.reshape(1, H // 2),
      Wf2, bf2.reshape(1, C))


def kernel(x, edge_index, W1, b1, ln1_w, ln1_b, W2, b2, ln2_w, ln2_b,
           Wg, bg, Wf1, bf1, Wf2, bf2):
    src2 = edge_index[0].reshape(NW, EPT)
    dst3 = edge_index[1].reshape(NW, NCHUNK, KS)

    deg2 = _sc_degree_kernel()(dst3).T
    xw = _tc1(x, W1)
    hs1 = _tc1s(xw, deg2)
    acc1 = _sc_scatter_kernel()(src2, dst3, hs1)
    hs2 = _tc2(hs1, acc1, deg2, b1, ln1_w, ln1_b, W2)
    acc2 = _sc_scatter_kernel()(src2, dst3, hs2)
    return _tc34(hs2, acc2, deg2, b2, ln2_w, ln2_b,
                 Wg, bg, Wf1, bf1, Wf2, bf2)


# speedup vs baseline: 1.0075x; 1.0075x over previous
import functools

import jax
import jax.numpy as jnp
from jax import lax
from jax.experimental import pallas as pl
from jax.experimental.pallas import tpu as pltpu
from jax.experimental.pallas import tpu_sc as plsc

N = 10000
E = 320000
D = 128
H = 128
C = 3

NC = 2
NS = 16
NW = NC * NS
NPAD = 10240
RPT = NPAD // NS
EPT = E // NW
K = 80

BLK = 1000

@functools.lru_cache(maxsize=None)
def _mesh():
    return plsc.VectorSubcoreMesh(core_axis_name="c", subcore_axis_name="s",
                                  num_cores=NC, num_subcores=NS)


def _sc_degree_body(dst_hbm, deg_hbm, idx_v, ones_v, buf_v,
                    deg_sh, isem, ssem0, ssem1):
    c = lax.axis_index("c")
    s = lax.axis_index("s")
    wid = s * NC + c
    pltpu.async_copy(dst_hbm.at[wid], idx_v, isem)
    zero16 = jnp.zeros((16,), jnp.float32)
    one16 = jnp.ones((16,), jnp.float32)
    for i in range(K // 16):
        ones_v[pl.ds(i * 16, 16)] = one16
    for i in range(RPT // 16):
        buf_v[pl.ds(i * 16, 16)] = zero16
    pltpu.sync_copy(buf_v, deg_sh.at[pl.ds(s * RPT, RPT)])
    pltpu.make_async_copy(dst_hbm.at[0], idx_v, isem).wait()
    plsc.subcore_barrier()

    ssem = (ssem0, ssem1)
    NCH = EPT // K

    def sstart(j, p):
        pltpu.async_copy(ones_v, deg_sh.at[idx_v.at[j]], ssem[p], add=True)

    def swait(p):
        pltpu.make_async_copy(ones_v, deg_sh.at[idx_v.at[0]], ssem[p]).wait()

    sstart(0, 0)
    sstart(1, 1)

    def pair(g, carry):
        j0 = 2 * g + 2
        swait(0)
        sstart(j0, 0)
        swait(1)
        sstart(j0 + 1, 1)
        return carry

    lax.fori_loop(0, (NCH - 3) // 2, pair, 0)
    swait(0)
    sstart(124, 0)
    swait(1)
    swait(0)
    plsc.subcore_barrier()
    pltpu.sync_copy(deg_sh.at[pl.ds(s * RPT, RPT)],
                    deg_hbm.at[c, pl.ds(s * RPT, RPT)])


@functools.lru_cache(maxsize=None)
def _sc_degree_kernel():
    return pl.kernel(
        _sc_degree_body,
        out_type=jax.ShapeDtypeStruct((NC, NPAD), jnp.float32),
        mesh=_mesh(),
        scratch_types=[
            pltpu.VMEM((EPT // K, K), jnp.int32),
            pltpu.VMEM((K,), jnp.float32),
            pltpu.VMEM((RPT,), jnp.float32),
            pltpu.VMEM_SHARED((NPAD,), jnp.float32),
            pltpu.SemaphoreType.DMA,
            pltpu.SemaphoreType.DMA,
            pltpu.SemaphoreType.DMA,
        ],
    )


KS = 80
NCHUNK = EPT // KS


def _sc_scatter_body(src_hbm, dst_hbm, hs_hbm, acc_hbm,
                     sidx_v, didx_v, rows0, rows1, acc_sh,
                     isem, gsem0, gsem1, ssem0, ssem1):
    c = lax.axis_index("c")
    s = lax.axis_index("s")
    wid = s * NC + c
    pltpu.async_copy(src_hbm.at[wid], sidx_v, isem)
    pltpu.async_copy(dst_hbm.at[wid], didx_v, isem)
    zero16 = jnp.zeros((16,), jnp.float32)

    def zrow(i, carry):
        for j in range(D // 16):
            rows0[i, pl.ds(j * 16, 16)] = zero16
        return carry

    lax.fori_loop(0, KS, zrow, 0)

    rows = (rows0, rows1)
    gsem = (gsem0, gsem1)
    ssem = (ssem0, ssem1)

    HK = KS // 2

    def gather(j, b):
        pltpu.async_copy(hs_hbm.at[sidx_v.at[pl.ds(j * KS, HK)]],
                         rows[b].at[pl.ds(0, HK)], gsem[b])
        pltpu.async_copy(hs_hbm.at[sidx_v.at[pl.ds(j * KS + HK, HK)]],
                         rows[b].at[pl.ds(HK, HK)], gsem[b])

    def gwait(b):
        for _ in range(2):
            pltpu.make_async_copy(hs_hbm.at[sidx_v.at[pl.ds(0, HK)]],
                                  rows[b].at[pl.ds(0, HK)], gsem[b]).wait()

    def sstart(j, b):
        pltpu.async_copy(rows[b], acc_sh.at[didx_v.at[j]], ssem[b], add=True)

    def swait(b):
        pltpu.make_async_copy(rows[0], acc_sh.at[didx_v.at[0]],
                              ssem[b]).wait()

    pltpu.make_async_copy(src_hbm.at[0], sidx_v, isem).wait()
    pltpu.make_async_copy(dst_hbm.at[0], didx_v, isem).wait()
    gather(0, 1)
    for t in range(RPT // KS):
        pltpu.async_copy(rows0, acc_sh.at[pl.ds(s * RPT + t * KS, KS)],
                         isem)
    for t in range(RPT // KS):
        pltpu.make_async_copy(rows0, acc_sh.at[pl.ds(0, KS)], isem).wait()
    plsc.subcore_barrier()

    def body(j, ph, do_swait=True, do_gather=True):
        if do_swait:
            swait(1 - ph)
        if do_gather:
            gather(j + 1, 1 - ph)
        gwait(ph)
        sstart(j, ph)

    body(0, 1, do_swait=False)

    def pair(g, carry):
        j0 = 2 * g + 1
        body(j0, 0)
        body(j0 + 1, 1)
        return carry

    lax.fori_loop(0, (NCHUNK - 3) // 2, pair, 0)
    body(123, 0)
    body(124, 1, do_gather=False)
    swait(1)
    plsc.subcore_barrier()
    pltpu.sync_copy(acc_sh.at[pl.ds(s * RPT, RPT)],
                    acc_hbm.at[c, pl.ds(s * RPT, RPT)])


@functools.lru_cache(maxsize=None)
def _sc_scatter_kernel():
    return pl.kernel(
        _sc_scatter_body,
        out_type=jax.ShapeDtypeStruct((NC, NPAD, D), jnp.float32),
        mesh=_mesh(),
        scratch_types=[
            pltpu.VMEM((EPT,), jnp.int32),
            pltpu.VMEM((NCHUNK, KS), jnp.int32),
            pltpu.VMEM((KS, D), jnp.float32),
            pltpu.VMEM((KS, D), jnp.float32),
            pltpu.VMEM_SHARED((NPAD, D), jnp.float32),
            pltpu.SemaphoreType.DMA,
            pltpu.SemaphoreType.DMA,
            pltpu.SemaphoreType.DMA,
            pltpu.SemaphoreType.DMA,
            pltpu.SemaphoreType.DMA,
        ],
    )


def _dinv_of(degb):
    return lax.rsqrt(degb[:, 0] + degb[:, 1] + 1.0)


def _tc1_body(x_ref, w_ref, deg_ref, out_ref):
    dinv = _dinv_of(deg_ref[...])
    h = jnp.dot(x_ref[...], w_ref[...], preferred_element_type=jnp.float32)
    out_ref[...] = h * dinv[:, None]


def _tc1(x, W1, deg2):
    return pl.pallas_call(
        _tc1_body,
        grid=(N // BLK,),
        in_specs=[
            pl.BlockSpec((BLK, D), lambda i: (i, 0)),
            pl.BlockSpec((D, H), lambda i: (0, 0)),
            pl.BlockSpec((BLK, NC), lambda i: (i, 0)),
        ],
        out_specs=pl.BlockSpec((BLK, H), lambda i: (i, 0)),
        out_shape=jax.ShapeDtypeStruct((N, H), jnp.float32),
    )(x, W1, deg2)


def _ln_relu(t, w, b):
    mu = jnp.mean(t, axis=-1, keepdims=True)
    var = jnp.mean((t - mu) ** 2, axis=-1, keepdims=True)
    t = (t - mu) * lax.rsqrt(var + 1e-5) * w + b
    return jnp.maximum(t, 0.0)


def _tc2_body(hs_ref, acc_ref, deg_ref, b_ref, lnw_ref, lnb_ref, w2_ref,
              out_ref):
    dinv = _dinv_of(deg_ref[...])
    a = acc_ref[0] + acc_ref[1]
    t = (hs_ref[...] + a) * dinv[:, None] + b_ref[...]
    t = _ln_relu(t, lnw_ref[...], lnb_ref[...])
    h2 = jnp.dot(t, w2_ref[...], preferred_element_type=jnp.float32)
    out_ref[...] = h2 * dinv[:, None]


def _tc2(hs1, acc1, deg2, b1, lnw, lnb, W2):
    return pl.pallas_call(
        _tc2_body,
        grid=(N // BLK,),
        in_specs=[
            pl.BlockSpec((BLK, H), lambda i: (i, 0)),
            pl.BlockSpec((NC, BLK, H), lambda i: (0, i, 0)),
            pl.BlockSpec((BLK, NC), lambda i: (i, 0)),
            pl.BlockSpec((1, H), lambda i: (0, 0)),
            pl.BlockSpec((1, H), lambda i: (0, 0)),
            pl.BlockSpec((1, H), lambda i: (0, 0)),
            pl.BlockSpec((H, H), lambda i: (0, 0)),
        ],
        out_specs=pl.BlockSpec((BLK, H), lambda i: (i, 0)),
        out_shape=jax.ShapeDtypeStruct((N, H), jnp.float32),
    )(hs1, acc1, deg2, b1.reshape(1, H), lnw.reshape(1, H),
      lnb.reshape(1, H), W2)


NB = N // BLK


def _tc34_body(hs_ref, acc_ref, deg_ref, b_ref, lnw_ref, lnb_ref,
               wg_ref, bg_ref, wf1_ref, bf1_ref, wf2_ref, bf2_ref,
               out_ref, h_scr, gsum_scr):
    i = pl.program_id(0)

    @pl.when(i < NB)
    def _():
        dinv = _dinv_of(deg_ref[...])
        a = acc_ref[0] + acc_ref[1]
        t = (hs_ref[...] + a) * dinv[:, None] + b_ref[...]
        t = _ln_relu(t, lnw_ref[...], lnb_ref[...])
        h_scr[pl.ds(i * BLK, BLK)] = t

        @pl.when(i == 0)
        def _():
            gsum_scr[...] = jnp.zeros_like(gsum_scr)

        gsum_scr[...] += jnp.sum(t, axis=0, keepdims=True)

    @pl.when(i >= NB)
    def _():
        g = gsum_scr[...] * (1.0 / N)
        tg = jnp.dot(g, wg_ref[...], preferred_element_type=jnp.float32) \
            + bg_ref[...]
        r = h_scr[pl.ds((i - NB) * BLK, BLK)] + tg
        r = jnp.dot(r, wf1_ref[...], preferred_element_type=jnp.float32) \
            + bf1_ref[...]
        r = jnp.maximum(r, 0.0)
        o = jnp.dot(r, wf2_ref[...], preferred_element_type=jnp.float32) \
            + bf2_ref[...]
        out_ref[...] = jnp.tanh(o)


def _tc34(hs2, acc2, deg2, b2, lnw, lnb, Wg, bg, Wf1, bf1, Wf2, bf2):
    lo = lambda i: (jnp.minimum(i, NB - 1), 0)
    lo3 = lambda i: (0, jnp.minimum(i, NB - 1), 0)
    z = lambda i: (0, 0)
    return pl.pallas_call(
        _tc34_body,
        grid=(2 * NB,),
        in_specs=[
            pl.BlockSpec((BLK, H), lo),
            pl.BlockSpec((NC, BLK, H), lo3),
            pl.BlockSpec((BLK, NC), lo),
            pl.BlockSpec((1, H), z),
            pl.BlockSpec((1, H), z),
            pl.BlockSpec((1, H), z),
            pl.BlockSpec((H, H), z),
            pl.BlockSpec((1, H), z),
            pl.BlockSpec((H, H // 2), z),
            pl.BlockSpec((1, H // 2), z),
            pl.BlockSpec((H // 2, C), z),
            pl.BlockSpec((1, C), z),
        ],
        out_specs=pl.BlockSpec((BLK, C), lambda i: (jnp.maximum(i - NB, 0),
                                                    0)),
        out_shape=jax.ShapeDtypeStruct((N, C), jnp.float32),
        scratch_shapes=[
            pltpu.VMEM((N, H), jnp.float32),
            pltpu.VMEM((1, H), jnp.float32),
        ],
    )(hs2, acc2, deg2, b2.reshape(1, H), lnw.reshape(1, H),
      lnb.reshape(1, H), Wg, bg.reshape(1, H), Wf1, bf1.reshape(1, H // 2),
      Wf2, bf2.reshape(1, C))


def kernel(x, edge_index, W1, b1, ln1_w, ln1_b, W2, b2, ln2_w, ln2_b,
           Wg, bg, Wf1, bf1, Wf2, bf2):
    src2 = edge_index[0].reshape(NW, EPT)
    dst3 = edge_index[1].reshape(NW, NCHUNK, KS)

    deg2 = _sc_degree_kernel()(dst3).T
    hs1 = _tc1(x, W1, deg2)
    acc1 = _sc_scatter_kernel()(src2, dst3, hs1)
    hs2 = _tc2(hs1, acc1, deg2, b1, ln1_w, ln1_b, W2)
    acc2 = _sc_scatter_kernel()(src2, dst3, hs2)
    return _tc34(hs2, acc2, deg2, b2, ln2_w, ln2_b,
                 Wg, bg, Wf1, bf1, Wf2, bf2)
